# baseline (speedup 1.0000x reference)
import functools

import jax
import jax.numpy as jnp
from jax import lax
from jax.experimental import pallas as pl
from jax.experimental.pallas import tpu as pltpu
from jax.experimental.pallas import tpu_sc as plsc

N_EXPERTS = 64
TOP_K = 8
TOK_BLOCK_TC = 512
KEY_SCALE = float(1 << 19)


def _tc_keys_body(x_ref, w_ref, out_ref):
    logits = lax.dot_general(
        x_ref[...],
        w_ref[...],
        dimension_numbers=(((1,), (0,)), ((), ())),
        preferred_element_type=jnp.float32,
    )
    fix = jnp.clip(
        logits * jnp.float32(KEY_SCALE), -33554000.0, 33554000.0
    ).astype(jnp.int32)
    e = lax.broadcasted_iota(jnp.int32, logits.shape, 1)
    out_ref[...] = lax.shift_left(fix, 6) | (jnp.int32(N_EXPERTS - 1) - e)


def _tc_keys(x, wt):
    t, h = x.shape
    return pl.pallas_call(
        _tc_keys_body,
        grid=(t // TOK_BLOCK_TC,),
        in_specs=[
            pl.BlockSpec((TOK_BLOCK_TC, h), lambda i: (i, 0)),
            pl.BlockSpec((h, N_EXPERTS), lambda i: (0, 0)),
        ],
        out_specs=pl.BlockSpec((TOK_BLOCK_TC, N_EXPERTS), lambda i: (i, 0)),
        out_shape=jax.ShapeDtypeStruct((t, N_EXPERTS), jnp.int32),
    )(x, wt)


def _sc_topk(keys):
    t = keys.shape[0]
    info = plsc.get_sparse_core_info()
    nc, ns, lanes = info.num_cores, info.num_subcores, info.num_lanes
    nw = nc * ns
    per_w = t // nw
    n_blocks = per_w // lanes
    blk = lanes * N_EXPERTS
    mesh = plsc.VectorSubcoreMesh(core_axis_name="c", subcore_axis_name="s")

    @functools.partial(
        pl.kernel,
        mesh=mesh,
        out_type=[
            jax.ShapeDtypeStruct((t, TOP_K), jnp.float32),
            jax.ShapeDtypeStruct((t, TOP_K), jnp.int32),
        ],
        scratch_types=[
            pltpu.VMEM((lanes, N_EXPERTS), jnp.int32),
            pltpu.VMEM((lanes, N_EXPERTS), jnp.int32),
            pltpu.VMEM((per_w, TOP_K), jnp.float32),
            pltpu.VMEM((per_w, TOP_K), jnp.int32),
            pltpu.SemaphoreType.DMA,
            pltpu.SemaphoreType.DMA,
        ],
        compiler_params=pltpu.CompilerParams(
            needs_layout_passes=False, use_tc_tiling_on_sc=False
        ),
    )
    def k(keys_hbm, outw_hbm, outi_hbm, lblk0, lblk1, wv, iv, sem0, sem1):
        wid = lax.axis_index("s") * nc + lax.axis_index("c")
        base = wid * per_w
        rows = lax.iota(jnp.int32, lanes)
        rows_scaled = rows * N_EXPERTS
        bot = jnp.full((lanes,), jnp.iinfo(jnp.int32).min, jnp.int32)
        sems = (sem0, sem1)
        bufs = (lblk0, lblk1)

        def src(b):
            return keys_hbm.at[pl.ds(base + b * lanes, lanes)]

        sort8_pairs = (
            (0, 1), (2, 3), (4, 5), (6, 7),
            (0, 2), (1, 3), (4, 6), (5, 7),
            (1, 2), (5, 6),
            (0, 4), (1, 5), (2, 6), (3, 7),
            (2, 4), (3, 5),
            (1, 2), (3, 4), (5, 6),
        )

        def sort8(g):
            for i, j in sort8_pairs:
                hi = jnp.maximum(g[i], g[j])
                g[j] = jnp.minimum(g[i], g[j])
                g[i] = hi
            return g

        def compute(buf, b):
            tkey = sort8([
                plsc.load_gather(buf, [rows, jnp.full((lanes,), e, jnp.int32)])
                for e in range(TOP_K)
            ])
            for g0 in range(TOP_K, N_EXPERTS, TOP_K):
                g = sort8([
                    plsc.load_gather(
                        buf, [rows, jnp.full((lanes,), g0 + q, jnp.int32)]
                    )
                    for q in range(TOP_K)
                ])
                h = [jnp.maximum(tkey[i], g[TOP_K - 1 - i]) for i in range(TOP_K)]
                for d in (4, 2, 1):
                    for i in range(TOP_K):
                        if (i // d) % 2 == 0:
                            hi = jnp.maximum(h[i], h[i + d])
                            h[i + d] = jnp.minimum(h[i], h[i + d])
                            h[i] = hi
                tkey = h
            idxs = [
                jnp.int32(N_EXPERTS - 1) - (tkey[j] & jnp.int32(63))
                for j in range(TOP_K)
            ]
            fixs = [lax.shift_right_arithmetic(tkey[j], 6) for j in range(TOP_K)]
            exps = [
                jnp.exp(
                    (fixs[j] - fixs[0]).astype(jnp.float32)
                    * jnp.float32(1.0 / KEY_SCALE)
                )
                for j in range(TOP_K)
            ]
            s = exps[0]
            for j in range(1, TOP_K):
                s = s + exps[j]
            r = 1.0 / s
            loc = b * lanes + rows
            for j in range(TOP_K):
                jv = jnp.full((lanes,), j, jnp.int32)
                plsc.store_scatter(wv, [loc, jv], exps[j] * r)
                plsc.store_scatter(iv, [loc, jv], idxs[j])

        pltpu.async_copy(src(0), lblk0, sem0)
        pltpu.async_copy(src(1), lblk1, sem1)

        def pair(g, carry):
            b0 = 2 * g
            for q in range(2):
                b = b0 + q
                buf, sem = bufs[q], sems[q]
                pltpu.make_async_copy(src(b), buf, sem).wait()
                compute(buf, b)
                nxt = jnp.minimum(b + 2, n_blocks - 1)
                pltpu.async_copy(src(nxt), buf, sem)
            return carry

        lax.fori_loop(0, n_blocks // 2, pair, 0)
        pltpu.make_async_copy(src(0), lblk0, sem0).wait()
        pltpu.make_async_copy(src(1), lblk1, sem1).wait()
        pltpu.sync_copy(wv, outw_hbm.at[pl.ds(base, per_w)])
        pltpu.sync_copy(iv, outi_hbm.at[pl.ds(base, per_w)])

    return k(keys)


def kernel(hidden_states, weight):
    b, s, h = hidden_states.shape
    x = hidden_states.reshape(-1, h)
    t = x.shape[0]
    keys = _tc_keys(x, weight.T)
    topw, topi = _sc_topk(keys)
    return topw, topi


# speedup vs baseline: 1.0619x; 1.0619x over previous
import functools

import jax
import jax.numpy as jnp
from jax import lax
from jax.experimental import pallas as pl
from jax.experimental.pallas import tpu as pltpu
from jax.experimental.pallas import tpu_sc as plsc

N_EXPERTS = 64
TOP_K = 8
TOK_BLOCK_TC = 512
TOK_BLOCK_FIN = 2048
KEY_SCALE = float(1 << 19)


def _tc_keys_body(x_ref, w_ref, out_ref):
    logits = lax.dot_general(
        x_ref[...],
        w_ref[...],
        dimension_numbers=(((1,), (1,)), ((), ())),
        preferred_element_type=jnp.float32,
    )
    fix = jnp.clip(
        logits * jnp.float32(KEY_SCALE), -33554000.0, 33554000.0
    ).astype(jnp.int32)
    e = lax.broadcasted_iota(jnp.int32, logits.shape, 1)
    keys = lax.shift_left(fix, 6) | (jnp.int32(N_EXPERTS - 1) - e)
    pad = jnp.zeros(keys.shape, jnp.int32)
    out_ref[...] = jnp.concatenate([keys, pad], axis=1)


def _tc_keys(x, w):
    t, h = x.shape
    return pl.pallas_call(
        _tc_keys_body,
        grid=(t // TOK_BLOCK_TC,),
        in_specs=[
            pl.BlockSpec((TOK_BLOCK_TC, h), lambda i: (i, 0)),
            pl.BlockSpec((N_EXPERTS, h), lambda i: (0, 0)),
        ],
        out_specs=pl.BlockSpec((TOK_BLOCK_TC, 2 * N_EXPERTS), lambda i: (i, 0)),
        out_shape=jax.ShapeDtypeStruct((t, 2 * N_EXPERTS), jnp.int32),
    )(x, w)


def _sc_select(keys):
    t = keys.shape[0]
    info = plsc.get_sparse_core_info()
    nc, ns, lanes = info.num_cores, info.num_subcores, info.num_lanes
    nw = nc * ns
    per_w = t // nw
    n_blocks = per_w // lanes
    mesh = plsc.VectorSubcoreMesh(core_axis_name="c", subcore_axis_name="s")

    @functools.partial(
        pl.kernel,
        mesh=mesh,
        out_type=jax.ShapeDtypeStruct((t, 2 * N_EXPERTS), jnp.int32),
        scratch_types=[
            pltpu.VMEM((lanes, 2 * N_EXPERTS), jnp.int32),
            pltpu.VMEM((lanes, 2 * N_EXPERTS), jnp.int32),
            pltpu.VMEM((per_w, 2 * N_EXPERTS), jnp.int32),
            pltpu.SemaphoreType.DMA,
            pltpu.SemaphoreType.DMA,
        ],
        compiler_params=pltpu.CompilerParams(
            needs_layout_passes=False, use_tc_tiling_on_sc=False
        ),
    )
    def k(keys_hbm, outk_hbm, lblk0, lblk1, kv, sem0, sem1):
        wid = lax.axis_index("s") * nc + lax.axis_index("c")
        base = wid * per_w
        rows = lax.iota(jnp.int32, lanes)
        sems = (sem0, sem1)
        bufs = (lblk0, lblk1)

        def src(b):
            return keys_hbm.at[pl.ds(base + b * lanes, lanes)]

        sort8_pairs = (
            (0, 1), (2, 3), (4, 5), (6, 7),
            (0, 2), (1, 3), (4, 6), (5, 7),
            (1, 2), (5, 6),
            (0, 4), (1, 5), (2, 6), (3, 7),
            (2, 4), (3, 5),
            (1, 2), (3, 4), (5, 6),
        )

        def sort8(g):
            for i, j in sort8_pairs:
                hi = jnp.maximum(g[i], g[j])
                g[j] = jnp.minimum(g[i], g[j])
                g[i] = hi
            return g

        def gather8(buf, g0):
            return [
                plsc.load_gather(buf, [rows, jnp.full((lanes,), g0 + q, jnp.int32)])
                for q in range(TOP_K)
            ]

        def compute(buf, b):
            tkey = sort8(gather8(buf, 0))
            for g0 in range(TOP_K, N_EXPERTS, TOP_K):
                g = sort8(gather8(buf, g0))
                h = [jnp.maximum(tkey[i], g[TOP_K - 1 - i]) for i in range(TOP_K)]
                for d in (4, 2, 1):
                    for i in range(TOP_K):
                        if (i // d) % 2 == 0:
                            hi = jnp.maximum(h[i], h[i + d])
                            h[i + d] = jnp.minimum(h[i], h[i + d])
                            h[i] = hi
                tkey = h
            loc = b * lanes + rows
            for j in range(TOP_K):
                jv = jnp.full((lanes,), j, jnp.int32)
                plsc.store_scatter(kv, [loc, jv], tkey[j])

        pltpu.async_copy(src(0), lblk0, sem0)
        pltpu.async_copy(src(1), lblk1, sem1)

        def pair(g, carry):
            b0 = 2 * g
            for q in range(2):
                b = b0 + q
                buf, sem = bufs[q], sems[q]
                pltpu.make_async_copy(src(b), buf, sem).wait()
                compute(buf, b)
                nxt = jnp.minimum(b + 2, n_blocks - 1)
                pltpu.async_copy(src(nxt), buf, sem)
            return carry

        lax.fori_loop(0, n_blocks // 2, pair, 0)
        pltpu.make_async_copy(src(0), lblk0, sem0).wait()
        pltpu.make_async_copy(src(1), lblk1, sem1).wait()
        pltpu.sync_copy(kv, outk_hbm.at[pl.ds(base, per_w)])

    return k(keys)


def _tc_finish_body(k_ref, w_ref, i_ref):
    k2 = k_ref[:, :TOP_K]
    i_ref[...] = jnp.int32(N_EXPERTS - 1) - (k2 & jnp.int32(63))
    fix = lax.shift_right_arithmetic(k2, 6)
    d = (fix - fix[:, 0:1]).astype(jnp.float32) * jnp.float32(1.0 / KEY_SCALE)
    e = jnp.exp(d)
    w_ref[...] = e / jnp.sum(e, axis=1, keepdims=True)


def _tc_finish(keys8, t):
    return pl.pallas_call(
        _tc_finish_body,
        grid=(t // TOK_BLOCK_FIN,),
        in_specs=[pl.BlockSpec((TOK_BLOCK_FIN, 2 * N_EXPERTS), lambda i: (i, 0))],
        out_specs=[
            pl.BlockSpec((TOK_BLOCK_FIN, TOP_K), lambda i: (i, 0)),
            pl.BlockSpec((TOK_BLOCK_FIN, TOP_K), lambda i: (i, 0)),
        ],
        out_shape=[
            jax.ShapeDtypeStruct((t, TOP_K), jnp.float32),
            jax.ShapeDtypeStruct((t, TOP_K), jnp.int32),
        ],
    )(keys8)


def kernel(hidden_states, weight):
    b, s, h = hidden_states.shape
    x = hidden_states.reshape(-1, h)
    t = x.shape[0]
    keys = _tc_keys(x, weight)
    keys8 = _sc_select(keys)
    topw, topi = _tc_finish(keys8, t)
    return topw, topi


# speedup vs baseline: 1.0851x; 1.0219x over previous
import functools

import jax
import jax.numpy as jnp
from jax import lax
from jax.experimental import pallas as pl
from jax.experimental.pallas import tpu as pltpu
from jax.experimental.pallas import tpu_sc as plsc

N_EXPERTS = 64
TOP_K = 8
TOK_BLOCK_TC = 512
TOK_BLOCK_FIN = 8192
KEY_SCALE = float(1 << 19)


def _tc_keys_body(x_ref, w_ref, out_ref):
    logits = lax.dot_general(
        x_ref[...],
        w_ref[...],
        dimension_numbers=(((1,), (1,)), ((), ())),
        preferred_element_type=jnp.float32,
    )
    fix = jnp.clip(
        logits * jnp.float32(KEY_SCALE), -33554000.0, 33554000.0
    ).astype(jnp.int32)
    e = lax.broadcasted_iota(jnp.int32, logits.shape, 1)
    keys = lax.shift_left(fix, 6) | (jnp.int32(N_EXPERTS - 1) - e)
    pad = jnp.zeros(keys.shape, jnp.int32)
    out_ref[...] = jnp.concatenate([keys, pad], axis=1)


def _tc_keys(x, w):
    t, h = x.shape
    return pl.pallas_call(
        _tc_keys_body,
        grid=(t // TOK_BLOCK_TC,),
        in_specs=[
            pl.BlockSpec((TOK_BLOCK_TC, h), lambda i: (i, 0)),
            pl.BlockSpec((N_EXPERTS, h), lambda i: (0, 0)),
        ],
        out_specs=pl.BlockSpec((TOK_BLOCK_TC, 2 * N_EXPERTS), lambda i: (i, 0)),
        out_shape=jax.ShapeDtypeStruct((t, 2 * N_EXPERTS), jnp.int32),
    )(x, w)


def _sc_select(keys):
    t = keys.shape[0]
    info = plsc.get_sparse_core_info()
    nc, ns, lanes = info.num_cores, info.num_subcores, info.num_lanes
    nw = nc * ns
    per_w = t // nw
    n_blocks = per_w // lanes
    mesh = plsc.VectorSubcoreMesh(core_axis_name="c", subcore_axis_name="s")

    @functools.partial(
        pl.kernel,
        mesh=mesh,
        out_type=jax.ShapeDtypeStruct((t, 2 * N_EXPERTS), jnp.int32),
        scratch_types=[
            pltpu.VMEM((lanes, 2 * N_EXPERTS), jnp.int32),
            pltpu.VMEM((lanes, 2 * N_EXPERTS), jnp.int32),
            pltpu.VMEM((per_w, 2 * N_EXPERTS), jnp.int32),
            pltpu.SemaphoreType.DMA,
            pltpu.SemaphoreType.DMA,
        ],
        compiler_params=pltpu.CompilerParams(
            needs_layout_passes=False, use_tc_tiling_on_sc=False
        ),
    )
    def k(keys_hbm, outk_hbm, lblk0, lblk1, kv, sem0, sem1):
        wid = lax.axis_index("s") * nc + lax.axis_index("c")
        base = wid * per_w
        rows = lax.iota(jnp.int32, lanes)
        sems = (sem0, sem1)
        bufs = (lblk0, lblk1)

        def src(b):
            return keys_hbm.at[pl.ds(base + b * lanes, lanes)]

        sort8_pairs = (
            (0, 1), (2, 3), (4, 5), (6, 7),
            (0, 2), (1, 3), (4, 6), (5, 7),
            (1, 2), (5, 6),
            (0, 4), (1, 5), (2, 6), (3, 7),
            (2, 4), (3, 5),
            (1, 2), (3, 4), (5, 6),
        )

        def sort8(g):
            for i, j in sort8_pairs:
                hi = jnp.maximum(g[i], g[j])
                g[j] = jnp.minimum(g[i], g[j])
                g[i] = hi
            return g

        def gather8(buf, g0):
            return [
                plsc.load_gather(buf, [rows, jnp.full((lanes,), g0 + q, jnp.int32)])
                for q in range(TOP_K)
            ]

        def compute(buf, b):
            tkey = sort8(gather8(buf, 0))
            for g0 in range(TOP_K, N_EXPERTS, TOP_K):
                g = sort8(gather8(buf, g0))
                h = [jnp.maximum(tkey[i], g[TOP_K - 1 - i]) for i in range(TOP_K)]
                for d in (4, 2, 1):
                    for i in range(TOP_K):
                        if (i // d) % 2 == 0:
                            hi = jnp.maximum(h[i], h[i + d])
                            h[i + d] = jnp.minimum(h[i], h[i + d])
                            h[i] = hi
                tkey = h
            loc = b * lanes + rows
            for j in range(TOP_K):
                jv = jnp.full((lanes,), j, jnp.int32)
                plsc.store_scatter(kv, [loc, jv], tkey[j])

        pltpu.async_copy(src(0), lblk0, sem0)
        pltpu.async_copy(src(1), lblk1, sem1)

        def pair(g, carry):
            b0 = 2 * g
            for q in range(2):
                b = b0 + q
                buf, sem = bufs[q], sems[q]
                pltpu.make_async_copy(src(b), buf, sem).wait()
                compute(buf, b)
                nxt = jnp.minimum(b + 2, n_blocks - 1)
                pltpu.async_copy(src(nxt), buf, sem)
            return carry

        lax.fori_loop(0, n_blocks // 2, pair, 0)
        pltpu.make_async_copy(src(0), lblk0, sem0).wait()
        pltpu.make_async_copy(src(1), lblk1, sem1).wait()
        pltpu.sync_copy(kv, outk_hbm.at[pl.ds(base, per_w)])

    return k(keys)


def _tc_finish_body(k_ref, w_ref, i_ref):
    k2 = k_ref[:, :TOP_K]
    i_ref[...] = jnp.int32(N_EXPERTS - 1) - (k2 & jnp.int32(63))
    fix = lax.shift_right_arithmetic(k2, 6)
    d = (fix - fix[:, 0:1]).astype(jnp.float32) * jnp.float32(1.0 / KEY_SCALE)
    e = jnp.exp(d)
    w_ref[...] = e / jnp.sum(e, axis=1, keepdims=True)


def _tc_finish(keys8, t):
    return pl.pallas_call(
        _tc_finish_body,
        grid=(t // TOK_BLOCK_FIN,),
        in_specs=[pl.BlockSpec((TOK_BLOCK_FIN, 2 * N_EXPERTS), lambda i: (i, 0))],
        out_specs=[
            pl.BlockSpec((TOK_BLOCK_FIN, TOP_K), lambda i: (i, 0)),
            pl.BlockSpec((TOK_BLOCK_FIN, TOP_K), lambda i: (i, 0)),
        ],
        out_shape=[
            jax.ShapeDtypeStruct((t, TOP_K), jnp.float32),
            jax.ShapeDtypeStruct((t, TOP_K), jnp.int32),
        ],
    )(keys8)


def kernel(hidden_states, weight):
    b, s, h = hidden_states.shape
    x = hidden_states.reshape(-1, h)
    t = x.shape[0]
    keys = _tc_keys(x, weight)
    keys8 = _sc_select(keys)
    topw, topi = _tc_finish(keys8, t)
    return topw, topi
